# baseline (speedup 1.0000x reference)
import jax
import jax.numpy as jnp
from jax.experimental import pallas as pl
from jax.experimental.pallas import tpu as pltpu

B, N, K = 256, 65536, 10
NB = 2048
GRID = N // NB


def _mrc_kernel(inputs_ref, rates_ref, out_ref, acc_ref, occ_ref):
    i = pl.program_id(0)

    r = rates_ref[...]
    l1 = jnp.maximum(jnp.sum(jnp.abs(r), axis=1, keepdims=True), 1e-12)
    m = jnp.max(r, axis=1, keepdims=True)
    lane = jax.lax.broadcasted_iota(jnp.int32, r.shape, 1)
    idx = jnp.min(jnp.where(r == m, lane, K), axis=1, keepdims=True)
    onehot = lane == idx
    assoc = jnp.where(onehot, m / l1, 0.0)

    part = jnp.dot(inputs_ref[...], assoc, preferred_element_type=jnp.float32)
    occ_part = jnp.sum(onehot.astype(jnp.float32), axis=0, keepdims=True)

    @pl.when(i == 0)
    def _():
        acc_ref[...] = jnp.zeros_like(acc_ref)
        occ_ref[...] = jnp.zeros_like(occ_ref)

    acc_ref[...] += part
    occ_ref[...] += occ_part

    @pl.when(i == GRID - 1)
    def _():
        occ = occ_ref[...]
        q = acc_ref[...] / occ
        q = jnp.where(jnp.isnan(q), 0.0, q)
        q = jnp.where(q == jnp.inf, 0.0, q)
        q = jnp.where(q == -jnp.inf, jnp.finfo(jnp.float32).min, q)
        out_ref[...] = q


@jax.jit
def kernel(inputs, rates):
    return pl.pallas_call(
        _mrc_kernel,
        grid=(GRID,),
        in_specs=[
            pl.BlockSpec((B, NB), lambda i: (0, i)),
            pl.BlockSpec((NB, K), lambda i: (i, 0)),
        ],
        out_specs=pl.BlockSpec((B, K), lambda i: (0, 0)),
        out_shape=jax.ShapeDtypeStruct((B, K), jnp.float32),
        scratch_shapes=[
            pltpu.VMEM((B, K), jnp.float32),
            pltpu.VMEM((1, K), jnp.float32),
        ],
        compiler_params=pltpu.CompilerParams(
            dimension_semantics=("arbitrary",),
        ),
    )(inputs, rates)


# speedup vs baseline: 2.0716x; 2.0716x over previous
import jax
import jax.numpy as jnp
from jax import lax
from jax.experimental import pallas as pl
from jax.experimental.pallas import tpu as pltpu

B, N, K = 256, 65536, 10
NB = 2048
GRID = N // NB


def _mrc_kernel(inputs_ref, rates_t_ref, out_ref, acc_ref, occ_ref):
    i = pl.program_id(0)

    rt = rates_t_ref[...]
    l1 = jnp.maximum(jnp.sum(jnp.abs(rt), axis=0, keepdims=True), 1e-12)
    m = jnp.max(rt, axis=0, keepdims=True)
    sub = lax.broadcasted_iota(jnp.int32, rt.shape, 0)
    idx = jnp.min(jnp.where(rt == m, sub, K), axis=0, keepdims=True)
    onehot = sub == idx
    s_t = jnp.where(onehot, m / l1, 0.0)

    part = lax.dot_general(
        inputs_ref[...], s_t,
        dimension_numbers=(((1,), (1,)), ((), ())),
        preferred_element_type=jnp.float32,
    )
    occ_part = jnp.sum(onehot.astype(jnp.float32), axis=1, keepdims=True)

    @pl.when(i == 0)
    def _():
        acc_ref[...] = jnp.zeros_like(acc_ref)
        occ_ref[...] = jnp.zeros_like(occ_ref)

    acc_ref[...] += part
    occ_ref[...] += occ_part

    @pl.when(i == GRID - 1)
    def _():
        occ = occ_ref[...].reshape(1, K)
        q = acc_ref[...] / occ
        q = jnp.where(jnp.isnan(q), 0.0, q)
        q = jnp.where(q == jnp.inf, 0.0, q)
        q = jnp.where(q == -jnp.inf, jnp.finfo(jnp.float32).min, q)
        out_ref[...] = q


@jax.jit
def kernel(inputs, rates):
    rates_t = rates.T
    return pl.pallas_call(
        _mrc_kernel,
        grid=(GRID,),
        in_specs=[
            pl.BlockSpec((B, NB), lambda i: (0, i)),
            pl.BlockSpec((K, NB), lambda i: (0, i)),
        ],
        out_specs=pl.BlockSpec((B, K), lambda i: (0, 0)),
        out_shape=jax.ShapeDtypeStruct((B, K), jnp.float32),
        scratch_shapes=[
            pltpu.VMEM((B, K), jnp.float32),
            pltpu.VMEM((K, 1), jnp.float32),
        ],
        compiler_params=pltpu.CompilerParams(
            dimension_semantics=("arbitrary",),
        ),
    )(inputs, rates_t)


# speedup vs baseline: 2.7030x; 1.3048x over previous
import jax
import jax.numpy as jnp
from jax import lax
from jax.experimental import pallas as pl
from jax.experimental.pallas import tpu as pltpu

B, N, K = 256, 65536, 10
NB = 4096
GRID = N // NB


def _mrc_kernel(inputs_ref, rates_t_ref, out_ref, acc_ref, occ_ref):
    i = pl.program_id(0)

    rt = rates_t_ref[...]
    l1 = jnp.maximum(jnp.sum(jnp.abs(rt), axis=0, keepdims=True), 1e-12)
    m = jnp.max(rt, axis=0, keepdims=True)
    sub = lax.broadcasted_iota(jnp.int32, rt.shape, 0)
    idx = jnp.min(jnp.where(rt == m, sub, K), axis=0, keepdims=True)
    onehot = sub == idx
    s_t = jnp.where(onehot, m / l1, 0.0)

    part = lax.dot_general(
        inputs_ref[...], s_t,
        dimension_numbers=(((1,), (1,)), ((), ())),
        preferred_element_type=jnp.float32,
    )
    occ_part = jnp.sum(onehot.astype(jnp.float32), axis=1, keepdims=True)

    @pl.when(i == 0)
    def _():
        acc_ref[...] = jnp.zeros_like(acc_ref)
        occ_ref[...] = jnp.zeros_like(occ_ref)

    acc_ref[...] += part
    occ_ref[...] += occ_part

    @pl.when(i == GRID - 1)
    def _():
        occ = occ_ref[...].reshape(1, K)
        q = acc_ref[...] / occ
        q = jnp.where(jnp.isnan(q), 0.0, q)
        q = jnp.where(q == jnp.inf, 0.0, q)
        q = jnp.where(q == -jnp.inf, jnp.finfo(jnp.float32).min, q)
        out_ref[...] = q


@jax.jit
def kernel(inputs, rates):
    rates_t = rates.T
    return pl.pallas_call(
        _mrc_kernel,
        grid=(GRID,),
        in_specs=[
            pl.BlockSpec((B, NB), lambda i: (0, i)),
            pl.BlockSpec((K, NB), lambda i: (0, i)),
        ],
        out_specs=pl.BlockSpec((B, K), lambda i: (0, 0)),
        out_shape=jax.ShapeDtypeStruct((B, K), jnp.float32),
        scratch_shapes=[
            pltpu.VMEM((B, K), jnp.float32),
            pltpu.VMEM((K, 1), jnp.float32),
        ],
        compiler_params=pltpu.CompilerParams(
            dimension_semantics=("arbitrary",),
        ),
    )(inputs, rates_t)


# speedup vs baseline: 3.0551x; 1.1303x over previous
import jax
import jax.numpy as jnp
from jax import lax
from jax.experimental import pallas as pl
from jax.experimental.pallas import tpu as pltpu

B, N, K = 256, 65536, 10
NB = 8192
GRID = N // NB


def _mrc_kernel(inputs_ref, rates_t_ref, out_ref, acc_ref, occ_ref):
    i = pl.program_id(0)

    rt = rates_t_ref[...]
    l1 = jnp.maximum(jnp.sum(jnp.abs(rt), axis=0, keepdims=True), 1e-12)
    m = jnp.max(rt, axis=0, keepdims=True)
    sub = lax.broadcasted_iota(jnp.int32, rt.shape, 0)
    idx = jnp.min(jnp.where(rt == m, sub, K), axis=0, keepdims=True)
    onehot = sub == idx
    s_t = jnp.where(onehot, m / l1, 0.0)

    part = lax.dot_general(
        inputs_ref[...], s_t,
        dimension_numbers=(((1,), (1,)), ((), ())),
        preferred_element_type=jnp.float32,
    )
    occ_part = jnp.sum(onehot.astype(jnp.float32), axis=1, keepdims=True)

    @pl.when(i == 0)
    def _():
        acc_ref[...] = jnp.zeros_like(acc_ref)
        occ_ref[...] = jnp.zeros_like(occ_ref)

    acc_ref[...] += part
    occ_ref[...] += occ_part

    @pl.when(i == GRID - 1)
    def _():
        occ = occ_ref[...].reshape(1, K)
        q = acc_ref[...] / occ
        q = jnp.where(jnp.isnan(q), 0.0, q)
        q = jnp.where(q == jnp.inf, 0.0, q)
        q = jnp.where(q == -jnp.inf, jnp.finfo(jnp.float32).min, q)
        out_ref[...] = q


@jax.jit
def kernel(inputs, rates):
    rates_t = rates.T
    return pl.pallas_call(
        _mrc_kernel,
        grid=(GRID,),
        in_specs=[
            pl.BlockSpec((B, NB), lambda i: (0, i)),
            pl.BlockSpec((K, NB), lambda i: (0, i)),
        ],
        out_specs=pl.BlockSpec((B, K), lambda i: (0, 0)),
        out_shape=jax.ShapeDtypeStruct((B, K), jnp.float32),
        scratch_shapes=[
            pltpu.VMEM((B, K), jnp.float32),
            pltpu.VMEM((K, 1), jnp.float32),
        ],
        compiler_params=pltpu.CompilerParams(
            dimension_semantics=("arbitrary",),
        ),
    )(inputs, rates_t)
